# Initial kernel scaffold; baseline (speedup 1.0000x reference)
import functools

import jax
import jax.numpy as jnp
from jax import lax
from jax.experimental import pallas as pl
from jax.experimental.pallas import tpu as pltpu
from jax.experimental.pallas import tpu_sc as plsc

N_NODES = 10000
N_EDGES = 320000
D = 128

NC = 2
NS = 16
NW = NC * NS
EPT = N_EDGES // NW
CHUNK = 80
NCHUNK = EPT // CHUNK
RPT = N_NODES // NS

_mesh = plsc.VectorSubcoreMesh(core_axis_name="c", subcore_axis_name="s")


def _sc_layer_body(x_hbm, col_hbm, row_hbm, ev_hbm, zeros_hbm, out_hbm,
                   colv, rowv, evv, msgs, acc, sem):
    c = lax.axis_index("c")
    s = lax.axis_index("s")
    wid = c * NS + s

    pltpu.sync_copy(zeros_hbm, acc.at[pl.ds(s * RPT, RPT)])
    plsc.subcore_barrier()

    base_e = wid * EPT

    def chunk_body(k, carry):
        e0 = base_e + k * CHUNK
        pltpu.sync_copy(col_hbm.at[pl.ds(e0, CHUNK)], colv)
        pltpu.sync_copy(row_hbm.at[pl.ds(e0, CHUNK)], rowv)
        pltpu.sync_copy(ev_hbm.at[pl.ds(e0, CHUNK)], evv)
        pltpu.async_copy(x_hbm.at[colv], msgs, sem).wait()

        def edge_body(i, carry2):
            sv = evv[i]
            for j in range(D // 16):
                sl = pl.ds(j * 16, 16)
                msgs[i, sl] = msgs[i, sl] * sv
            return carry2

        lax.fori_loop(0, CHUNK, edge_body, 0)
        pltpu.sync_copy(msgs, acc.at[rowv], add=True)
        return carry

    lax.fori_loop(0, NCHUNK, chunk_body, 0)
    plsc.subcore_barrier()
    pltpu.sync_copy(acc.at[pl.ds(s * RPT, RPT)],
                    out_hbm.at[c, pl.ds(s * RPT, RPT)])


_sc_layer = pl.kernel(
    _sc_layer_body,
    mesh=_mesh,
    out_type=jax.ShapeDtypeStruct((NC, N_NODES, D), jnp.float32),
    scratch_types=[
        pltpu.VMEM((CHUNK,), jnp.int32),
        pltpu.VMEM((CHUNK,), jnp.int32),
        pltpu.VMEM((CHUNK,), jnp.float32),
        pltpu.VMEM((CHUNK, D), jnp.float32),
        pltpu.VMEM_SHARED((N_NODES, D), jnp.float32),
        pltpu.SemaphoreType.DMA,
    ],
)


def _combine_body(p_ref, o_ref):
    o_ref[...] = p_ref[0] + p_ref[1]


_combine = pl.pallas_call(
    _combine_body,
    grid=(10,),
    in_specs=[pl.BlockSpec((NC, 1000, D), lambda i: (0, i, 0))],
    out_specs=pl.BlockSpec((1000, D), lambda i: (i, 0)),
    out_shape=jax.ShapeDtypeStruct((N_NODES, D), jnp.float32),
)


def _final_body(x_ref, w_ref, b_ref, p1_ref, p2_ref, o_ref):
    emb0 = lax.dot_general(x_ref[...], w_ref[...], (((1,), (1,)), ((), ())),
                           preferred_element_type=jnp.float32)
    o_ref[...] = (emb0 + b_ref[...] + p1_ref[0] + p1_ref[1]
                  + p2_ref[0] + p2_ref[1]) * (1.0 / 3.0)


_final = pl.pallas_call(
    _final_body,
    grid=(10,),
    in_specs=[
        pl.BlockSpec((1000, D), lambda i: (i, 0)),
        pl.BlockSpec((D, D), lambda i: (0, 0)),
        pl.BlockSpec((1, D), lambda i: (0, 0)),
        pl.BlockSpec((NC, 1000, D), lambda i: (0, i, 0)),
        pl.BlockSpec((NC, 1000, D), lambda i: (0, i, 0)),
    ],
    out_specs=pl.BlockSpec((1000, D), lambda i: (i, 0)),
    out_shape=jax.ShapeDtypeStruct((N_NODES, D), jnp.float32),
)


def kernel(all_emb, edge_index, edge_values, W, b):
    row = edge_index[0].astype(jnp.int32)
    col = edge_index[1].astype(jnp.int32)
    zeros = jnp.zeros((RPT, D), jnp.float32)
    p1 = _sc_layer(all_emb, col, row, edge_values, zeros)
    x1 = _combine(p1)
    p2 = _sc_layer(x1, col, row, edge_values, zeros)
    out = _final(all_emb, W, b.reshape(1, D), p1, p2)
    return out


# speedup vs baseline: 4.1748x; 4.1748x over previous
import functools

import jax
import jax.numpy as jnp
from jax import lax
from jax.experimental import pallas as pl
from jax.experimental.pallas import tpu as pltpu
from jax.experimental.pallas import tpu_sc as plsc

N_NODES = 10000
N_EDGES = 320000
D = 128

NC = 2
NS = 16
NW = NC * NS
EPT = N_EDGES // NW
CHUNK = 80
NCHUNK = EPT // CHUNK
RPT = 624
RPT_LAST = N_NODES - (NS - 1) * RPT

_mesh = plsc.VectorSubcoreMesh(core_axis_name="c", subcore_axis_name="s")


def _sc_layer_body(x_hbm, col_hbm, row_hbm, ev_hbm, zeros_hbm, out_hbm,
                   colv, rowv, evv, msgs, acc, sem):
    c = lax.axis_index("c")
    s = lax.axis_index("s")
    wid = c * NS + s

    @pl.when(s < NS - 1)
    def _():
        pltpu.sync_copy(zeros_hbm.at[pl.ds(0, RPT)], acc.at[pl.ds(s * RPT, RPT)])

    @pl.when(s == NS - 1)
    def _():
        pltpu.sync_copy(zeros_hbm, acc.at[pl.ds((NS - 1) * RPT, RPT_LAST)])

    plsc.subcore_barrier()

    base_e = wid * EPT

    def chunk_body(k, carry):
        e0 = base_e + k * CHUNK
        pltpu.sync_copy(col_hbm.at[pl.ds(e0, CHUNK)], colv)
        pltpu.sync_copy(row_hbm.at[pl.ds(e0, CHUNK)], rowv)
        pltpu.sync_copy(ev_hbm.at[pl.ds(e0, CHUNK)], evv)
        pltpu.async_copy(x_hbm.at[colv], msgs, sem).wait()

        def group_body(g, carry2):
            ev16 = evv[pl.ds(g * 16, 16)]
            for l in range(16):
                sv = ev16[l]
                i = g * 16 + l
                for j in range(D // 16):
                    sl = pl.ds(j * 16, 16)
                    msgs[i, sl] = msgs[i, sl] * sv
            return carry2

        lax.fori_loop(0, CHUNK // 16, group_body, 0)
        pltpu.sync_copy(msgs, acc.at[rowv], add=True)
        return carry

    lax.fori_loop(0, NCHUNK, chunk_body, 0)
    plsc.subcore_barrier()

    @pl.when(s < NS - 1)
    def _():
        pltpu.sync_copy(acc.at[pl.ds(s * RPT, RPT)],
                        out_hbm.at[c, pl.ds(s * RPT, RPT)])

    @pl.when(s == NS - 1)
    def _():
        pltpu.sync_copy(acc.at[pl.ds((NS - 1) * RPT, RPT_LAST)],
                        out_hbm.at[c, pl.ds((NS - 1) * RPT, RPT_LAST)])


_sc_layer = pl.kernel(
    _sc_layer_body,
    mesh=_mesh,
    out_type=jax.ShapeDtypeStruct((NC, N_NODES, D), jnp.float32),
    scratch_types=[
        pltpu.VMEM((CHUNK,), jnp.int32),
        pltpu.VMEM((CHUNK,), jnp.int32),
        pltpu.VMEM((CHUNK,), jnp.float32),
        pltpu.VMEM((CHUNK, D), jnp.float32),
        pltpu.VMEM_SHARED((N_NODES, D), jnp.float32),
        pltpu.SemaphoreType.DMA,
    ],
)


def _combine_body(p_ref, o_ref):
    o_ref[...] = p_ref[0] + p_ref[1]


_combine = pl.pallas_call(
    _combine_body,
    grid=(10,),
    in_specs=[pl.BlockSpec((NC, 1000, D), lambda i: (0, i, 0))],
    out_specs=pl.BlockSpec((1000, D), lambda i: (i, 0)),
    out_shape=jax.ShapeDtypeStruct((N_NODES, D), jnp.float32),
)


def _final_body(x_ref, w_ref, b_ref, p1_ref, p2_ref, o_ref):
    emb0 = lax.dot_general(x_ref[...], w_ref[...], (((1,), (1,)), ((), ())),
                           preferred_element_type=jnp.float32)
    o_ref[...] = (emb0 + b_ref[...] + p1_ref[0] + p1_ref[1]
                  + p2_ref[0] + p2_ref[1]) * (1.0 / 3.0)


_final = pl.pallas_call(
    _final_body,
    grid=(10,),
    in_specs=[
        pl.BlockSpec((1000, D), lambda i: (i, 0)),
        pl.BlockSpec((D, D), lambda i: (0, 0)),
        pl.BlockSpec((1, D), lambda i: (0, 0)),
        pl.BlockSpec((NC, 1000, D), lambda i: (0, i, 0)),
        pl.BlockSpec((NC, 1000, D), lambda i: (0, i, 0)),
    ],
    out_specs=pl.BlockSpec((1000, D), lambda i: (i, 0)),
    out_shape=jax.ShapeDtypeStruct((N_NODES, D), jnp.float32),
)


def kernel(all_emb, edge_index, edge_values, W, b):
    row = edge_index[0].astype(jnp.int32)
    col = edge_index[1].astype(jnp.int32)
    zeros = jnp.zeros((RPT_LAST, D), jnp.float32)
    p1 = _sc_layer(all_emb, col, row, edge_values, zeros)
    x1 = _combine(p1)
    p2 = _sc_layer(x1, col, row, edge_values, zeros)
    out = _final(all_emb, W, b.reshape(1, D), p1, p2)
    return out
